# Initial kernel scaffold; baseline (speedup 1.0000x reference)
import jax
import jax.numpy as jnp
from jax.experimental import pallas as pl


def kernel(x, W, bias):
    raise NotImplementedError("write your pallas kernel here")


# speedup vs baseline: 34.4254x; 34.4254x over previous
import functools

import jax
import jax.numpy as jnp
from jax.experimental import pallas as pl
from jax.experimental.pallas import tpu as pltpu

K = 32
BISECT_ITERS = 30


def _gnn_kernel(x_ref, w_ref, b_ref, o_ref):
    xb = x_ref[0]
    c = xb.shape[0]
    fp = jnp.dot(w_ref[...], xb, preferred_element_type=jnp.float32)
    fp = fp + b_ref[...]
    featT = fp[:c, :]
    posT = fp[c:, :]
    nrm2 = jnp.sum(posT * posT, axis=0, keepdims=True)
    posn = posT * jax.lax.rsqrt(jnp.maximum(nrm2, 1e-24))
    sim = jax.lax.dot_general(
        posn, posn,
        dimension_numbers=(((0,), (0,)), ((), ())),
        preferred_element_type=jnp.float32,
    )
    mx = jnp.max(sim, axis=0, keepdims=True)
    lo0 = jnp.full_like(mx, -1.0)
    hi0 = mx

    def body(_, carry):
        lo, hi = carry
        mid = 0.5 * (lo + hi)
        cnt = jnp.sum(jnp.where(sim >= mid, 1.0, 0.0), axis=0, keepdims=True)
        ge = cnt >= K
        return jnp.where(ge, mid, lo), jnp.where(ge, hi, mid)

    lo, hi = jax.lax.fori_loop(0, BISECT_ITERS, body, (lo0, hi0))
    e = jnp.where(sim >= lo, jnp.exp(sim - mx), 0.0)
    z = jnp.sum(e, axis=0, keepdims=True)
    st = e * (1.0 / z)
    o_ref[0] = jnp.dot(featT, st, preferred_element_type=jnp.float32)


def kernel(x, W, bias):
    b, c, h, w = x.shape
    n = h * w
    xr = x.reshape(b, c, n)
    out = pl.pallas_call(
        _gnn_kernel,
        grid=(b,),
        in_specs=[
            pl.BlockSpec((1, c, n), lambda i: (i, 0, 0)),
            pl.BlockSpec((2 * c, c), lambda i: (0, 0)),
            pl.BlockSpec((2 * c, 1), lambda i: (0, 0)),
        ],
        out_specs=pl.BlockSpec((1, c, n), lambda i: (i, 0, 0)),
        out_shape=jax.ShapeDtypeStruct((b, c, n), jnp.float32),
    )(xr, W, bias.reshape(2 * c, 1))
    return out.reshape(b, c, h, w)


# speedup vs baseline: 36.2012x; 1.0516x over previous
import jax
import jax.numpy as jnp
from jax.experimental import pallas as pl

K = 32
BISECT_ITERS = 24
GROUPS = 32


def _gnn_kernel(x_ref, xb16_ref, wp_ref, wf_ref, bp_ref, bf_ref, o_ref):
    xb = x_ref[0]
    xb16 = xb16_ref[0]
    n = xb.shape[1]
    posT = jnp.dot(wp_ref[...], xb, preferred_element_type=jnp.float32)
    posT = posT + bp_ref[...]
    nrm2 = jnp.sum(posT * posT, axis=0, keepdims=True)
    posn = posT * jax.lax.rsqrt(jnp.maximum(nrm2, 1e-24))
    sim = jax.lax.dot_general(
        posn, posn,
        dimension_numbers=(((0,), (0,)), ((), ())),
        preferred_element_type=jnp.float32,
    )
    featT = jnp.dot(wf_ref[...], xb16, preferred_element_type=jnp.float32)
    featT = (featT + bf_ref[...]).astype(jnp.bfloat16)

    g = n // GROUPS
    mx = None
    gmin = None
    for i in range(GROUPS):
        bm = jnp.max(sim[i * g:(i + 1) * g, :], axis=0, keepdims=True)
        mx = bm if mx is None else jnp.maximum(mx, bm)
        gmin = bm if gmin is None else jnp.minimum(gmin, bm)

    def body(_, carry):
        lo, hi = carry
        mid = 0.5 * (lo + hi)
        cnt = jnp.sum(jnp.where(sim >= mid, 1.0, 0.0), axis=0, keepdims=True)
        ge = cnt >= K
        return jnp.where(ge, mid, lo), jnp.where(ge, hi, mid)

    lo, hi = jax.lax.fori_loop(0, BISECT_ITERS, body, (gmin, mx))
    e = jnp.where(sim >= lo, jnp.exp(sim - mx), 0.0)
    z = jnp.sum(e, axis=0, keepdims=True)
    eb = e.astype(jnp.bfloat16)
    acc = jnp.dot(featT, eb, preferred_element_type=jnp.float32)
    o_ref[0] = acc * (1.0 / z)


def kernel(x, W, bias):
    b, c, h, w = x.shape
    n = h * w
    xr = x.reshape(b, c, n)
    xr16 = xr.astype(jnp.bfloat16)
    wf = W[:c].astype(jnp.bfloat16)
    wp = W[c:]
    bf = bias[:c].reshape(c, 1)
    bp = bias[c:].reshape(c, 1)
    out = pl.pallas_call(
        _gnn_kernel,
        grid=(b,),
        in_specs=[
            pl.BlockSpec((1, c, n), lambda i: (i, 0, 0)),
            pl.BlockSpec((1, c, n), lambda i: (i, 0, 0)),
            pl.BlockSpec((c, c), lambda i: (0, 0)),
            pl.BlockSpec((c, c), lambda i: (0, 0)),
            pl.BlockSpec((c, 1), lambda i: (0, 0)),
            pl.BlockSpec((c, 1), lambda i: (0, 0)),
        ],
        out_specs=pl.BlockSpec((1, c, n), lambda i: (i, 0, 0)),
        out_shape=jax.ShapeDtypeStruct((b, c, n), jnp.float32),
    )(xr, xr16, wp, wf, bp, bf)
    return out.reshape(b, c, h, w)


# speedup vs baseline: 39.0301x; 1.0781x over previous
import jax
import jax.numpy as jnp
from jax.experimental import pallas as pl

K = 32
BISECT_ITERS = 24
GROUPS = 32


def _gnn_kernel(x_ref, xb16_ref, wp_ref, wf_ref, bp_ref, bf_ref, o_ref):
    xb = x_ref[0]
    xb16 = xb16_ref[0]
    n = xb.shape[1]
    posT = jnp.dot(wp_ref[...], xb, preferred_element_type=jnp.float32)
    posT = posT + bp_ref[...]
    nrm2 = jnp.sum(posT * posT, axis=0, keepdims=True)
    posn = posT * jax.lax.rsqrt(jnp.maximum(nrm2, 1e-24))
    sim = jax.lax.dot_general(
        posn, posn,
        dimension_numbers=(((0,), (0,)), ((), ())),
        preferred_element_type=jnp.float32,
    )
    featT = jnp.dot(wf_ref[...], xb16, preferred_element_type=jnp.float32)
    featT = (featT + bf_ref[...]).astype(jnp.bfloat16)

    g = n // GROUPS
    mx = None
    gmin = None
    for i in range(GROUPS):
        bm = jnp.max(sim[i * g:(i + 1) * g, :], axis=0, keepdims=True)
        mx = bm if mx is None else jnp.maximum(mx, bm)
        gmin = bm if gmin is None else jnp.minimum(gmin, bm)

    ones_row = jnp.ones((1, n), jnp.float32)

    def body(_, carry):
        lo, hi = carry
        mid = 0.5 * (lo + hi)
        maskf = jnp.where(sim >= mid, 1.0, 0.0)
        cnt = jnp.dot(ones_row, maskf, preferred_element_type=jnp.float32)
        ge = cnt >= K
        return jnp.where(ge, mid, lo), jnp.where(ge, hi, mid)

    lo, hi = jax.lax.fori_loop(0, BISECT_ITERS, body, (gmin, mx))
    e = jnp.where(sim >= lo, jnp.exp(sim - mx), 0.0)
    z = jnp.sum(e, axis=0, keepdims=True)
    eb = e.astype(jnp.bfloat16)
    acc = jnp.dot(featT, eb, preferred_element_type=jnp.float32)
    o_ref[0] = acc * (1.0 / z)


def kernel(x, W, bias):
    b, c, h, w = x.shape
    n = h * w
    xr = x.reshape(b, c, n)
    xr16 = xr.astype(jnp.bfloat16)
    wf = W[:c].astype(jnp.bfloat16)
    wp = W[c:]
    bf = bias[:c].reshape(c, 1)
    bp = bias[c:].reshape(c, 1)
    out = pl.pallas_call(
        _gnn_kernel,
        grid=(b,),
        in_specs=[
            pl.BlockSpec((1, c, n), lambda i: (i, 0, 0)),
            pl.BlockSpec((1, c, n), lambda i: (i, 0, 0)),
            pl.BlockSpec((c, c), lambda i: (0, 0)),
            pl.BlockSpec((c, c), lambda i: (0, 0)),
            pl.BlockSpec((c, 1), lambda i: (0, 0)),
            pl.BlockSpec((c, 1), lambda i: (0, 0)),
        ],
        out_specs=pl.BlockSpec((1, c, n), lambda i: (i, 0, 0)),
        out_shape=jax.ShapeDtypeStruct((b, c, n), jnp.float32),
    )(xr, xr16, wp, wf, bp, bf)
    return out.reshape(b, c, h, w)


# speedup vs baseline: 40.2794x; 1.0320x over previous
import jax
import jax.numpy as jnp
from jax.experimental import pallas as pl

K = 32
BISECT_ITERS = 22
GROUPS = 32


def _gnn_kernel(x_ref, xb16_ref, wp_ref, wf_ref, bp_ref, bf_ref, o_ref):
    xb = x_ref[0]
    xb16 = xb16_ref[0]
    n = xb.shape[1]
    posT = jnp.dot(wp_ref[...], xb, preferred_element_type=jnp.float32)
    posT = posT + bp_ref[...]
    nrm2 = jnp.sum(posT * posT, axis=0, keepdims=True)
    posn = posT * jax.lax.rsqrt(jnp.maximum(nrm2, 1e-24))
    sim = jax.lax.dot_general(
        posn, posn,
        dimension_numbers=(((0,), (0,)), ((), ())),
        preferred_element_type=jnp.float32,
    )
    featT = jnp.dot(wf_ref[...], xb16, preferred_element_type=jnp.float32)
    featT = (featT + bf_ref[...]).astype(jnp.bfloat16)

    g = n // GROUPS
    br = jax.lax.broadcasted_iota(jnp.int32, (g, n), 0)
    bc = jax.lax.broadcasted_iota(jnp.int32, (g, n), 1)
    mx_off = None
    gmin = None
    for i in range(GROUPS):
        blk = jnp.where(br + (i * g) == bc, -2.0, sim[i * g:(i + 1) * g, :])
        bm = jnp.max(blk, axis=0, keepdims=True)
        mx_off = bm if mx_off is None else jnp.maximum(mx_off, bm)
        gmin = bm if gmin is None else jnp.minimum(gmin, bm)
    mx = jnp.maximum(mx_off, 1.0)

    ones_row = jnp.ones((1, n), jnp.float32)

    def body(_, carry):
        lo, hi = carry
        mid = 0.5 * (lo + hi)
        maskf = jnp.where(sim >= mid, 1.0, 0.0)
        cnt = jnp.dot(ones_row, maskf, preferred_element_type=jnp.float32)
        ge = cnt >= K
        return jnp.where(ge, mid, lo), jnp.where(ge, hi, mid)

    lo, hi = jax.lax.fori_loop(0, BISECT_ITERS, body, (gmin, mx_off))
    e = jnp.where(sim >= lo, jnp.exp(sim - mx), 0.0)
    z = jnp.sum(e, axis=0, keepdims=True)
    eb = e.astype(jnp.bfloat16)
    acc = jnp.dot(featT, eb, preferred_element_type=jnp.float32)
    o_ref[0] = acc * (1.0 / z)


def kernel(x, W, bias):
    b, c, h, w = x.shape
    n = h * w
    xr = x.reshape(b, c, n)
    xr16 = xr.astype(jnp.bfloat16)
    wf = W[:c].astype(jnp.bfloat16)
    wp = W[c:]
    bf = bias[:c].reshape(c, 1)
    bp = bias[c:].reshape(c, 1)
    out = pl.pallas_call(
        _gnn_kernel,
        grid=(b,),
        in_specs=[
            pl.BlockSpec((1, c, n), lambda i: (i, 0, 0)),
            pl.BlockSpec((1, c, n), lambda i: (i, 0, 0)),
            pl.BlockSpec((c, c), lambda i: (0, 0)),
            pl.BlockSpec((c, c), lambda i: (0, 0)),
            pl.BlockSpec((c, 1), lambda i: (0, 0)),
            pl.BlockSpec((c, 1), lambda i: (0, 0)),
        ],
        out_specs=pl.BlockSpec((1, c, n), lambda i: (i, 0, 0)),
        out_shape=jax.ShapeDtypeStruct((b, c, n), jnp.float32),
    )(xr, xr16, wp, wf, bp, bf)
    return out.reshape(b, c, h, w)
